# baseline (speedup 1.0000x reference)
import jax
import jax.numpy as jnp
from jax.experimental import pallas as pl
from jax.experimental.pallas import tpu as pltpu

N_TOKENS = 8192
IN_DIM = 768
OUT_DIM = 768
NUM_EXPERTS = 8
TOP_K = 2

TILE_N = 1024


def _moe_tile_kernel(x_ref, wg_ref, bg_ref, we_hbm, be_ref, out_ref,
                     we_vmem, we_sems):
    i = pl.program_id(0)

    @pl.when(i == 0)
    def _():
        for e in range(NUM_EXPERTS):
            pltpu.make_async_copy(we_hbm.at[e], we_vmem.at[e],
                                  we_sems.at[e]).start()

    x_t = x_ref[:]
    scores = jnp.dot(x_t, wg_ref[:], preferred_element_type=jnp.float32)
    scores = scores + bg_ref[:]

    t = x_t.shape[0]
    e_iota = jax.lax.broadcasted_iota(jnp.int32, (t, NUM_EXPERTS), 1)

    m1 = jnp.max(scores, axis=-1, keepdims=True)
    i1 = jnp.min(jnp.where(scores == m1, e_iota, NUM_EXPERTS), axis=-1,
                 keepdims=True)
    oh1 = (e_iota == i1)

    scores2 = jnp.where(oh1, -jnp.inf, scores)
    m2 = jnp.max(scores2, axis=-1, keepdims=True)
    i2 = jnp.min(jnp.where(scores2 == m2, e_iota, NUM_EXPERTS), axis=-1,
                 keepdims=True)
    oh2 = (e_iota == i2)

    d = jnp.exp(m2 - m1)
    w1 = 1.0 / (1.0 + d)
    w2 = d / (1.0 + d)
    wvec = w1 * oh1.astype(jnp.float32) + w2 * oh2.astype(jnp.float32)

    acc = jnp.zeros((t, OUT_DIM), dtype=jnp.float32)
    for e in range(NUM_EXPERTS):
        @pl.when(i == 0)
        def _():
            pltpu.make_async_copy(we_hbm.at[e], we_vmem.at[e],
                                  we_sems.at[e]).wait()
        y = jnp.dot(x_t, we_vmem[e], preferred_element_type=jnp.float32)
        y = y + be_ref[e][None, :]
        acc = acc + wvec[:, e][:, None] * y
    out_ref[:] = acc


@jax.jit
def kernel(x, W_g, b_g, W_e, b_e):
    n = x.shape[0]
    grid = (n // TILE_N,)
    return pl.pallas_call(
        _moe_tile_kernel,
        grid=grid,
        in_specs=[
            pl.BlockSpec((TILE_N, IN_DIM), lambda i: (i, 0)),
            pl.BlockSpec((IN_DIM, NUM_EXPERTS), lambda i: (0, 0)),
            pl.BlockSpec((NUM_EXPERTS,), lambda i: (0,)),
            pl.BlockSpec(memory_space=pl.ANY),
            pl.BlockSpec((NUM_EXPERTS, OUT_DIM), lambda i: (0, 0)),
        ],
        out_specs=pl.BlockSpec((TILE_N, OUT_DIM), lambda i: (i, 0)),
        out_shape=jax.ShapeDtypeStruct((n, OUT_DIM), jnp.float32),
        scratch_shapes=[
            pltpu.VMEM((NUM_EXPERTS, IN_DIM, OUT_DIM), jnp.float32),
            pltpu.SemaphoreType.DMA((NUM_EXPERTS,)),
        ],
    )(x, W_g, b_g, W_e, b_e)


# speedup vs baseline: 1.3800x; 1.3800x over previous
import jax
import jax.numpy as jnp
from jax.experimental import pallas as pl

N_TOKENS = 8192
IN_DIM = 768
OUT_DIM = 768
NUM_EXPERTS = 8
TOP_K = 2

TILE_N = 1024


def _moe_tile_kernel(x_ref, wg_ref, bg_ref, we_ref, be_ref, out_ref):
    x_t = x_ref[:]
    scores = jnp.dot(x_t, wg_ref[:], preferred_element_type=jnp.float32)
    scores = scores + bg_ref[:]

    t = x_t.shape[0]
    e_iota = jax.lax.broadcasted_iota(jnp.int32, (t, NUM_EXPERTS), 1)

    m1 = jnp.max(scores, axis=-1, keepdims=True)
    i1 = jnp.min(jnp.where(scores == m1, e_iota, NUM_EXPERTS), axis=-1,
                 keepdims=True)
    oh1 = (e_iota == i1)

    scores2 = jnp.where(oh1, -jnp.inf, scores)
    m2 = jnp.max(scores2, axis=-1, keepdims=True)
    i2 = jnp.min(jnp.where(scores2 == m2, e_iota, NUM_EXPERTS), axis=-1,
                 keepdims=True)
    oh2 = (e_iota == i2)

    d = jnp.exp(m2 - m1)
    w1 = 1.0 / (1.0 + d)
    w2 = d / (1.0 + d)
    wvec = w1 * oh1.astype(jnp.float32) + w2 * oh2.astype(jnp.float32)

    acc = jnp.zeros((t, OUT_DIM), dtype=jnp.float32)
    for e in range(NUM_EXPERTS):
        y = jnp.dot(x_t, we_ref[e], preferred_element_type=jnp.float32)
        y = y + be_ref[e][None, :]
        acc = acc + wvec[:, e][:, None] * y
    out_ref[:] = acc


@jax.jit
def kernel(x, W_g, b_g, W_e, b_e):
    n = x.shape[0]
    grid = (n // TILE_N,)
    return pl.pallas_call(
        _moe_tile_kernel,
        grid=grid,
        in_specs=[
            pl.BlockSpec((TILE_N, IN_DIM), lambda i: (i, 0)),
            pl.BlockSpec((IN_DIM, NUM_EXPERTS), lambda i: (0, 0)),
            pl.BlockSpec((NUM_EXPERTS,), lambda i: (0,)),
            pl.BlockSpec((NUM_EXPERTS, IN_DIM, OUT_DIM), lambda i: (0, 0, 0)),
            pl.BlockSpec((NUM_EXPERTS, OUT_DIM), lambda i: (0, 0)),
        ],
        out_specs=pl.BlockSpec((TILE_N, OUT_DIM), lambda i: (i, 0)),
        out_shape=jax.ShapeDtypeStruct((n, OUT_DIM), jnp.float32),
    )(x, W_g, b_g, W_e, b_e)
